# Initial kernel scaffold; baseline (speedup 1.0000x reference)
import jax
import jax.numpy as jnp
from jax.experimental import pallas as pl


def kernel(x, edge_index, edge_attr, W_in, b_in, W_msg, W_edge, W_upd, b_upd, ln_g, ln_b, W_out1, b_out1, W_out2, b_out2):
    raise NotImplementedError("write your pallas kernel here")


# speedup vs baseline: 3.1281x; 3.1281x over previous
import functools

import jax
import jax.numpy as jnp
from jax import lax
from jax.experimental import pallas as pl
from jax.experimental.pallas import tpu as pltpu
from jax.experimental.pallas import tpu_sc as plsc

_DH = 128
_NW = 32
_NS = 16
_C = 128
_BLK = 1024


def _ceil_to(a, m):
    return (a + m - 1) // m * m


def _sc_edge_body(np_pad, epw, nm_hbm, src_hbm, dst_hbm, ea_hbm, w_hbm,
                  out_hbm, agg_sh, src_v, dst_v, ea_v, rows_v, w_v, sem):
    cid = lax.axis_index("c")
    sid = lax.axis_index("s")
    wid = cid * _NS + sid
    rows_per_tile = np_pad // _NS

    def _zrow(r, _):
        for j in range(_DH // 16):
            rows_v[r, pl.ds(j * 16, 16)] = jnp.zeros((16,), jnp.float32)
        return 0
    lax.fori_loop(0, _C, _zrow, 0)
    for k in range(rows_per_tile // _C):
        pltpu.sync_copy(rows_v, agg_sh.at[pl.ds(sid * rows_per_tile + k * _C, _C)])

    pltpu.sync_copy(w_hbm, w_v)
    wregs = [w_v[pl.ds(j * 16, 16)] for j in range(_DH // 16)]

    plsc.subcore_barrier()

    base = wid * epw

    def _chunk(ci, _):
        off = base + ci * _C
        pltpu.sync_copy(src_hbm.at[pl.ds(off, _C)], src_v)
        pltpu.sync_copy(dst_hbm.at[pl.ds(off, _C)], dst_v)
        pltpu.sync_copy(ea_hbm.at[pl.ds(off, _C)], ea_v)
        pltpu.async_copy(nm_hbm.at[src_v], rows_v, sem).wait()

        def _grp(g, _):
            ea_g = ea_v[pl.ds(g * 16, 16)]
            for l in range(16):
                e = g * 16 + l
                eav = jnp.full((16,), ea_g[l], jnp.float32)
                for j in range(_DH // 16):
                    v = rows_v[e, pl.ds(j * 16, 16)]
                    rows_v[e, pl.ds(j * 16, 16)] = jnp.maximum(
                        v + eav * wregs[j], 0.0)
            return 0
        lax.fori_loop(0, _C // 16, _grp, 0)

        pltpu.sync_copy(rows_v, agg_sh.at[dst_v], add=True)
        return 0

    lax.fori_loop(0, epw // _C, _chunk, 0)

    plsc.subcore_barrier()

    for k in range(rows_per_tile // _C):
        r0 = sid * rows_per_tile + k * _C
        pltpu.sync_copy(agg_sh.at[pl.ds(r0, _C)], out_hbm.at[cid, pl.ds(r0, _C)])


def _sc_deg_body(np_pad, epw, dst_hbm, out_hbm, deg_sh, dst_v, ones_v, sem):
    del sem
    cid = lax.axis_index("c")
    sid = lax.axis_index("s")
    wid = cid * _NS + sid
    rows_per_tile = np_pad // _NS

    def _orow(r, _):
        for j in range(_DH // 16):
            ones_v[r, pl.ds(j * 16, 16)] = jnp.zeros((16,), jnp.float32)
        return 0
    lax.fori_loop(0, _C, _orow, 0)
    for k in range(rows_per_tile // _C):
        pltpu.sync_copy(ones_v, deg_sh.at[pl.ds(sid * rows_per_tile + k * _C, _C)])

    def _orow1(r, _):
        for j in range(_DH // 16):
            ones_v[r, pl.ds(j * 16, 16)] = jnp.full((16,), 1.0, jnp.float32)
        return 0
    lax.fori_loop(0, _C, _orow1, 0)

    plsc.subcore_barrier()

    base = wid * epw

    def _chunk(ci, _):
        off = base + ci * _C
        pltpu.sync_copy(dst_hbm.at[pl.ds(off, _C)], dst_v)
        pltpu.sync_copy(ones_v, deg_sh.at[dst_v], add=True)
        return 0

    lax.fori_loop(0, epw // _C, _chunk, 0)

    plsc.subcore_barrier()

    for k in range(rows_per_tile // _C):
        r0 = sid * rows_per_tile + k * _C
        pltpu.sync_copy(deg_sh.at[pl.ds(r0, _C)], out_hbm.at[cid, pl.ds(r0, _C)])


def _make_sc_edge(np_pad, epw):
    mesh = plsc.VectorSubcoreMesh(core_axis_name="c", subcore_axis_name="s")
    return pl.kernel(
        functools.partial(_sc_edge_body, np_pad, epw),
        out_type=jax.ShapeDtypeStruct((2, np_pad, _DH), jnp.float32),
        mesh=mesh,
        scratch_types=[
            pltpu.VMEM_SHARED((np_pad, _DH), jnp.float32),
            pltpu.VMEM((_C,), jnp.int32),
            pltpu.VMEM((_C,), jnp.int32),
            pltpu.VMEM((_C,), jnp.float32),
            pltpu.VMEM((_C, _DH), jnp.float32),
            pltpu.VMEM((_DH,), jnp.float32),
            pltpu.SemaphoreType.DMA,
        ],
    )


def _make_sc_deg(np_pad, epw):
    mesh = plsc.VectorSubcoreMesh(core_axis_name="c", subcore_axis_name="s")
    return pl.kernel(
        functools.partial(_sc_deg_body, np_pad, epw),
        out_type=jax.ShapeDtypeStruct((2, np_pad, _DH), jnp.float32),
        mesh=mesh,
        scratch_types=[
            pltpu.VMEM_SHARED((np_pad, _DH), jnp.float32),
            pltpu.VMEM((_C,), jnp.int32),
            pltpu.VMEM((_C, _DH), jnp.float32),
            pltpu.SemaphoreType.DMA,
        ],
    )


def _pre_body(x_ref, wi_ref, bi_ref, wm_ref, degp_ref, h_ref, nm_ref, invd_ref):
    h = jnp.dot(x_ref[...], wi_ref[...], preferred_element_type=jnp.float32)
    h = h + bi_ref[...]
    h_ref[...] = h
    nm_ref[...] = jnp.dot(h, wm_ref[...], preferred_element_type=jnp.float32)
    deg = degp_ref[0] + degp_ref[1]
    invd_ref[...] = 1.0 / jnp.maximum(deg, 1.0)


def _upd_body(aggp_ref, invd_ref, hb_ref, wu_ref, bu_ref, g_ref, b_ref,
              wm_ref, hb_out_ref, nm_ref):
    agg = (aggp_ref[0] + aggp_ref[1]) * invd_ref[...]
    upd = jnp.dot(agg, wu_ref[...], preferred_element_type=jnp.float32)
    upd = jnp.maximum(upd + bu_ref[...], 0.0)
    t = hb_ref[...] + upd
    mu = jnp.mean(t, axis=-1, keepdims=True)
    var = jnp.mean((t - mu) ** 2, axis=-1, keepdims=True)
    hbn = (t - mu) * lax.rsqrt(var + 1e-5) * g_ref[...] + b_ref[...]
    hb_out_ref[...] = hbn
    nm_ref[...] = jnp.dot(hbn, wm_ref[...], preferred_element_type=jnp.float32)


def _final_body(h_ref, h1_ref, h2_ref, h3_ref, w1_ref, b1_ref, w2_ref, b2_ref,
                out_ref):
    e = jnp.dot(h_ref[...], w1_ref[pl.ds(0, _DH)],
                preferred_element_type=jnp.float32)
    e = e + jnp.dot(h1_ref[...], w1_ref[pl.ds(_DH, _DH)],
                    preferred_element_type=jnp.float32)
    e = e + jnp.dot(h2_ref[...], w1_ref[pl.ds(2 * _DH, _DH)],
                    preferred_element_type=jnp.float32)
    e = e + jnp.dot(h3_ref[...], w1_ref[pl.ds(3 * _DH, _DH)],
                    preferred_element_type=jnp.float32)
    e = jnp.maximum(e + b1_ref[...], 0.0)
    out_ref[...] = jnp.dot(e, w2_ref[...],
                           preferred_element_type=jnp.float32) + b2_ref[...]


def _row_spec(np_pad):
    return pl.BlockSpec((_BLK, _DH), lambda i: (i, 0))


def _full_spec(shape):
    nd = len(shape)
    return pl.BlockSpec(shape, lambda i: (0,) * nd)


def _tc_pre(np_pad, x, wi, bi, wm, degp):
    grid = np_pad // _BLK
    return pl.pallas_call(
        _pre_body,
        grid=(grid,),
        in_specs=[
            _row_spec(np_pad),
            _full_spec((_DH, _DH)),
            _full_spec((1, _DH)),
            _full_spec((_DH, _DH)),
            pl.BlockSpec((2, _BLK, _DH), lambda i: (0, i, 0)),
        ],
        out_specs=[_row_spec(np_pad)] * 3,
        out_shape=[jax.ShapeDtypeStruct((np_pad, _DH), jnp.float32)] * 3,
    )(x, wi, bi, wm, degp)


def _tc_upd(np_pad, aggp, invd, hb, wu, bu, g, b, wm):
    grid = np_pad // _BLK
    return pl.pallas_call(
        _upd_body,
        grid=(grid,),
        in_specs=[
            pl.BlockSpec((2, _BLK, _DH), lambda i: (0, i, 0)),
            _row_spec(np_pad),
            _row_spec(np_pad),
            _full_spec((_DH, _DH)),
            _full_spec((1, _DH)),
            _full_spec((1, _DH)),
            _full_spec((1, _DH)),
            _full_spec((_DH, _DH)),
        ],
        out_specs=[_row_spec(np_pad)] * 2,
        out_shape=[jax.ShapeDtypeStruct((np_pad, _DH), jnp.float32)] * 2,
    )(aggp, invd, hb, wu, bu, g, b, wm)


def _tc_final(np_pad, h, h1, h2, h3, w1, b1, w2, b2):
    grid = np_pad // _BLK
    return pl.pallas_call(
        _final_body,
        grid=(grid,),
        in_specs=[
            _row_spec(np_pad),
            _row_spec(np_pad),
            _row_spec(np_pad),
            _row_spec(np_pad),
            _full_spec((4 * _DH, _DH)),
            _full_spec((1, _DH)),
            _full_spec((_DH, _DH)),
            _full_spec((1, _DH)),
        ],
        out_specs=_row_spec(np_pad),
        out_shape=jax.ShapeDtypeStruct((np_pad, _DH), jnp.float32),
    )(h, h1, h2, h3, w1, b1, w2, b2)


def kernel(x, edge_index, edge_attr, W_in, b_in, W_msg, W_edge, W_upd, b_upd,
           ln_g, ln_b, W_out1, b_out1, W_out2, b_out2):
    n, df = x.shape
    e = edge_index.shape[1]
    l_layers = W_msg.shape[0]
    blocks = l_layers // 2

    np_pad = _ceil_to(n + 1, _NS * _C)
    e_pad = _ceil_to(e, _NW * _C)
    epw = e_pad // _NW

    x_pad = jnp.pad(x, ((0, np_pad - n), (0, 0)))
    src = jnp.pad(edge_index[0], (0, e_pad - e))
    dst = jnp.pad(edge_index[1], (0, e_pad - e), constant_values=n)
    ea = jnp.pad(edge_attr[:, 0], (0, e_pad - e))

    sc_edge = _make_sc_edge(np_pad, epw)
    sc_deg = _make_sc_deg(np_pad, epw)

    degp = sc_deg(dst)
    h, nm, invd = _tc_pre(np_pad, x_pad, W_in,
                          b_in.reshape(1, _DH), W_msg[0], degp)

    outs = [h]
    hb = h
    for i in range(l_layers):
        aggp = sc_edge(nm, src, dst, ea, W_edge[i].reshape(_DH))
        hb, nm = _tc_upd(np_pad, aggp, invd, hb,
                         W_upd[i], b_upd[i].reshape(1, _DH),
                         ln_g[i].reshape(1, _DH), ln_b[i].reshape(1, _DH),
                         W_msg[(i + 1) % l_layers])
        if i % 2 == 1:
            outs.append(hb)

    out = _tc_final(np_pad, outs[0], outs[1], outs[2], outs[3],
                    W_out1, b_out1.reshape(1, _DH), W_out2,
                    b_out2.reshape(1, _DH))
    return out[:n]


# speedup vs baseline: 3.1939x; 1.0210x over previous
import functools

import jax
import jax.numpy as jnp
from jax import lax
from jax.experimental import pallas as pl
from jax.experimental.pallas import tpu as pltpu
from jax.experimental.pallas import tpu_sc as plsc

_DH = 128
_FH = 64
_NW = 32
_NS = 16
_CE = 80
_CD = 128
_BLK = 1024


def _ceil_to(a, m):
    return (a + m - 1) // m * m


def _sc_edge_body(np_pad, nchunks, nm_hbm, src_hbm, dst_hbm, ea_hbm, w_hbm,
                  out_hbm, agg_sh, rows, srcs, dsts, eas, w_v, sg, se, ss):
    cid = lax.axis_index("c")
    sid = lax.axis_index("s")
    wid = cid * _NS + sid
    rows_per_tile = np_pad // _NS

    pltpu.sync_copy(w_hbm, w_v)
    wregs = [w_v[pl.ds(j * 16, 16)] for j in range(_DH // 16)]

    def _zrow(r, _):
        for j in range(_DH // 16):
            rows[0][r, pl.ds(j * 16, 16)] = jnp.zeros((16,), jnp.float32)
        return 0
    lax.fori_loop(0, _CE, _zrow, 0)
    r = 0
    while r < rows_per_tile:
        size = min(_CE, rows_per_tile - r)
        src_buf = rows[0] if size == _CE else rows[0].at[pl.ds(0, size)]
        pltpu.sync_copy(src_buf, agg_sh.at[pl.ds(sid * rows_per_tile + r, size)])
        r += size

    plsc.subcore_barrier()

    def _start_edata(ci, b):
        pltpu.async_copy(src_hbm.at[wid, ci], srcs[b], se[b])
        pltpu.async_copy(dst_hbm.at[wid, ci], dsts[b], se[b])
        pltpu.async_copy(ea_hbm.at[wid, ci], eas[b], se[b])

    def _wait_edata(b):
        pltpu.make_async_copy(src_hbm.at[0, 0], srcs[b], se[b]).wait()
        pltpu.make_async_copy(dst_hbm.at[0, 0], dsts[b], se[b]).wait()
        pltpu.make_async_copy(ea_hbm.at[0, 0], eas[b], se[b]).wait()

    def _start_gather(b):
        pltpu.async_copy(nm_hbm.at[srcs[b]], rows[b], sg[b])

    def _wait_gather(b):
        pltpu.make_async_copy(nm_hbm.at[srcs[b]], rows[b], sg[b]).wait()

    def _start_scatter(b):
        pltpu.async_copy(rows[b], agg_sh.at[dsts[b]], ss[b], add=True)

    def _wait_scatter(b):
        pltpu.make_async_copy(rows[b], agg_sh.at[dsts[b]], ss[b]).wait()

    def _compute(b):
        buf = rows[b]
        ea = eas[b]

        def _grp(g, _):
            ea_g = ea[pl.ds(g * 16, 16)]
            for l in range(16):
                e = g * 16 + l
                eav = jnp.full((16,), ea_g[l], jnp.float32)
                for j in range(_DH // 16):
                    v = buf[e, pl.ds(j * 16, 16)]
                    buf[e, pl.ds(j * 16, 16)] = jnp.maximum(
                        v + eav * wregs[j], 0.0)
            return 0
        lax.fori_loop(0, _CE // 16, _grp, 0)

    def _step(ci, b, first=False, second=False, tail1=False, tail2=False):
        _wait_gather(b)
        if not (first or second):
            _wait_scatter((b + 2) % 4)
        if not tail2:
            _wait_edata((b + 1) % 4)
            _start_gather((b + 1) % 4)
        if not (tail1 or tail2):
            _start_edata(ci + 2, (b + 2) % 4)
        _compute(b)
        _start_scatter(b)

    assert nchunks % 4 == 0 and nchunks >= 8
    _start_edata(0, 0)
    _start_edata(1, 1)
    _wait_edata(0)
    _start_gather(0)
    _step(0, 0, first=True)
    _step(1, 1, second=True)

    def _quad(p, _):
        ci = 2 + 4 * p
        _step(ci, 2)
        _step(ci + 1, 3)
        _step(ci + 2, 0)
        _step(ci + 3, 1)
        return 0
    lax.fori_loop(0, (nchunks - 4) // 4, _quad, 0)

    _step(nchunks - 2, 2, tail1=True)
    _step(nchunks - 1, 3, tail2=True)
    _wait_scatter(2)
    _wait_scatter(3)

    plsc.subcore_barrier()

    r = 0
    while r < rows_per_tile:
        size = min(_CE, rows_per_tile - r)
        r0 = sid * rows_per_tile + r
        pltpu.sync_copy(agg_sh.at[pl.ds(r0, size)], out_hbm.at[cid, pl.ds(r0, size)])
        r += size


def _sc_deg_body(np_pad, epw, dst_hbm, out_hbm, deg_sh, dst_v, ones_v, sem):
    del sem
    cid = lax.axis_index("c")
    sid = lax.axis_index("s")
    wid = cid * _NS + sid
    rows_per_tile = np_pad // _NS

    def _orow(r, _):
        for j in range(_DH // 16):
            ones_v[r, pl.ds(j * 16, 16)] = jnp.zeros((16,), jnp.float32)
        return 0
    lax.fori_loop(0, _CD, _orow, 0)
    for k in range(rows_per_tile // _CD):
        pltpu.sync_copy(ones_v,
                        deg_sh.at[pl.ds(sid * rows_per_tile + k * _CD, _CD)])

    def _orow1(r, _):
        for j in range(_DH // 16):
            ones_v[r, pl.ds(j * 16, 16)] = jnp.full((16,), 1.0, jnp.float32)
        return 0
    lax.fori_loop(0, _CD, _orow1, 0)

    plsc.subcore_barrier()

    base = wid * epw

    def _chunk(ci, _):
        pltpu.sync_copy(dst_hbm.at[pl.ds(base + ci * _CD, _CD)], dst_v)
        pltpu.sync_copy(ones_v, deg_sh.at[dst_v], add=True)
        return 0

    lax.fori_loop(0, epw // _CD, _chunk, 0)

    plsc.subcore_barrier()

    for k in range(rows_per_tile // _CD):
        r0 = sid * rows_per_tile + k * _CD
        pltpu.sync_copy(deg_sh.at[pl.ds(r0, _CD)], out_hbm.at[cid, pl.ds(r0, _CD)])


def _make_sc_edge(np_pad, nchunks):
    mesh = plsc.VectorSubcoreMesh(core_axis_name="c", subcore_axis_name="s")

    def body(nm_hbm, src_hbm, dst_hbm, ea_hbm, w_hbm, out_hbm, agg_sh,
             r0, r1, r2, r3, s0, s1, s2, s3, d0, d1, d2, d3,
             a0, a1, a2, a3, w_v,
             sg0, sg1, sg2, sg3, se0, se1, se2, se3, ss0, ss1, ss2, ss3):
        _sc_edge_body(np_pad, nchunks, nm_hbm, src_hbm, dst_hbm, ea_hbm,
                      w_hbm, out_hbm, agg_sh, (r0, r1, r2, r3),
                      (s0, s1, s2, s3), (d0, d1, d2, d3), (a0, a1, a2, a3),
                      w_v, (sg0, sg1, sg2, sg3), (se0, se1, se2, se3),
                      (ss0, ss1, ss2, ss3))

    return pl.kernel(
        body,
        out_type=jax.ShapeDtypeStruct((2, np_pad, _DH), jnp.float32),
        mesh=mesh,
        scratch_types=(
            [pltpu.VMEM_SHARED((np_pad, _DH), jnp.float32)]
            + [pltpu.VMEM((_CE, _DH), jnp.float32) for _ in range(4)]
            + [pltpu.VMEM((_CE,), jnp.int32) for _ in range(8)]
            + [pltpu.VMEM((_CE,), jnp.float32) for _ in range(4)]
            + [pltpu.VMEM((_DH,), jnp.float32)]
            + [pltpu.SemaphoreType.DMA] * 12
        ),
    )


def _make_sc_deg(np_pad, nchunks):
    mesh = plsc.VectorSubcoreMesh(core_axis_name="c", subcore_axis_name="s")
    return pl.kernel(
        functools.partial(_sc_deg_body, np_pad, nchunks * _CD),
        out_type=jax.ShapeDtypeStruct((2, np_pad, _DH), jnp.float32),
        mesh=mesh,
        scratch_types=[
            pltpu.VMEM_SHARED((np_pad, _DH), jnp.float32),
            pltpu.VMEM((_CD,), jnp.int32),
            pltpu.VMEM((_CD, _DH), jnp.float32),
            pltpu.SemaphoreType.DMA,
        ],
    )


def _pre_body(x_ref, wi_ref, bi_ref, wm_ref, degp_ref, h_ref, nm_ref,
              invd_ref):
    h = jnp.dot(x_ref[...], wi_ref[...], preferred_element_type=jnp.float32)
    h = h + bi_ref[...]
    h_ref[...] = h
    nm_ref[...] = jnp.dot(h, wm_ref[...], preferred_element_type=jnp.float32)
    deg = degp_ref[0] + degp_ref[1]
    invd_ref[...] = 1.0 / jnp.maximum(deg, 1.0)


def _upd_body(aggp_ref, invd_ref, hb_ref, wu_ref, bu_ref, g_ref, b_ref,
              wm_ref, hb_out_ref, nm_ref):
    agg = (aggp_ref[0] + aggp_ref[1]) * invd_ref[...]
    upd = jnp.dot(agg, wu_ref[...], preferred_element_type=jnp.float32)
    upd = jnp.maximum(upd + bu_ref[...], 0.0)
    t = hb_ref[...] + upd
    mu = jnp.mean(t, axis=-1, keepdims=True)
    var = jnp.mean((t - mu) ** 2, axis=-1, keepdims=True)
    hbn = (t - mu) * lax.rsqrt(var + 1e-5) * g_ref[...] + b_ref[...]
    hb_out_ref[...] = hbn
    nm_ref[...] = jnp.dot(hbn, wm_ref[...], preferred_element_type=jnp.float32)


def _final_body(h_ref, h1_ref, h2_ref, h3_ref, w1_ref, b1_ref, w2_ref, b2_ref,
                out_ref):
    e = jnp.dot(h_ref[...], w1_ref[pl.ds(0, _DH)],
                preferred_element_type=jnp.float32)
    e = e + jnp.dot(h1_ref[...], w1_ref[pl.ds(_DH, _DH)],
                    preferred_element_type=jnp.float32)
    e = e + jnp.dot(h2_ref[...], w1_ref[pl.ds(2 * _DH, _DH)],
                    preferred_element_type=jnp.float32)
    e = e + jnp.dot(h3_ref[...], w1_ref[pl.ds(3 * _DH, _DH)],
                    preferred_element_type=jnp.float32)
    e = jnp.maximum(e + b1_ref[...], 0.0)
    out_ref[...] = jnp.dot(e, w2_ref[...],
                           preferred_element_type=jnp.float32) + b2_ref[...]


def _row_spec():
    return pl.BlockSpec((_BLK, _DH), lambda i: (i, 0))


def _part_spec(width):
    return pl.BlockSpec((2, _BLK, width), lambda i: (0, i, 0))


def _full_spec(shape):
    nd = len(shape)
    return pl.BlockSpec(shape, lambda i: (0,) * nd)


def _tc_pre(np_pad, x, wi, bi, wm, degp):
    grid = np_pad // _BLK
    return pl.pallas_call(
        _pre_body,
        grid=(grid,),
        in_specs=[
            _row_spec(),
            _full_spec((_DH, _DH)),
            _full_spec((1, _DH)),
            _full_spec((_DH, _DH)),
            _part_spec(_DH),
        ],
        out_specs=[_row_spec(), _row_spec(), _row_spec()],
        out_shape=[jax.ShapeDtypeStruct((np_pad, _DH), jnp.float32)] * 3,
    )(x, wi, bi, wm, degp)


def _tc_upd(np_pad, aggp, invd, hb, wu, bu, g, b, wm):
    grid = np_pad // _BLK
    return pl.pallas_call(
        _upd_body,
        grid=(grid,),
        in_specs=[
            _part_spec(_DH),
            _row_spec(),
            _row_spec(),
            _full_spec((_DH, _DH)),
            _full_spec((1, _DH)),
            _full_spec((1, _DH)),
            _full_spec((1, _DH)),
            _full_spec((_DH, _DH)),
        ],
        out_specs=[_row_spec(), _row_spec()],
        out_shape=[jax.ShapeDtypeStruct((np_pad, _DH), jnp.float32)] * 2,
    )(aggp, invd, hb, wu, bu, g, b, wm)


def _tc_final(np_pad, h, h1, h2, h3, w1, b1, w2, b2):
    grid = np_pad // _BLK
    return pl.pallas_call(
        _final_body,
        grid=(grid,),
        in_specs=[
            _row_spec(),
            _row_spec(),
            _row_spec(),
            _row_spec(),
            _full_spec((4 * _DH, _DH)),
            _full_spec((1, _DH)),
            _full_spec((_DH, _DH)),
            _full_spec((1, _DH)),
        ],
        out_specs=_row_spec(),
        out_shape=jax.ShapeDtypeStruct((np_pad, _DH), jnp.float32),
    )(h, h1, h2, h3, w1, b1, w2, b2)


def kernel(x, edge_index, edge_attr, W_in, b_in, W_msg, W_edge, W_upd, b_upd,
           ln_g, ln_b, W_out1, b_out1, W_out2, b_out2):
    n, df = x.shape
    e = edge_index.shape[1]
    l_layers = W_msg.shape[0]

    np_pad = _ceil_to(n + 1, _NS * _CD)

    nch_e = _ceil_to(-(-e // (_NW * _CE)), 4)
    e_pe = _NW * nch_e * _CE
    nch_d = _ceil_to(-(-e // (_NW * _CD)), 8)
    e_pd = _NW * nch_d * _CD

    x_pad = jnp.pad(x, ((0, np_pad - n), (0, 0)))
    src = jnp.pad(edge_index[0], (0, e_pe - e))
    dst = jnp.pad(edge_index[1], (0, e_pe - e), constant_values=n)
    eattr = jnp.pad(edge_attr[:, 0], (0, e_pe - e)).reshape(_NW, nch_e, _CE)
    src = src.reshape(_NW, nch_e, _CE)
    dst = dst.reshape(_NW, nch_e, _CE)

    dst_d = jnp.pad(edge_index[1], (0, e_pd - e), constant_values=n)

    sc_edge = _make_sc_edge(np_pad, nch_e)
    sc_deg = _make_sc_deg(np_pad, nch_d)

    degp = sc_deg(dst_d)
    h, nm, invd = _tc_pre(np_pad, x_pad, W_in,
                          b_in.reshape(1, _DH), W_msg[0], degp)

    outs = [h]
    hb = h
    for i in range(l_layers):
        aggp = sc_edge(nm, src, dst, eattr, W_edge[i].reshape(_DH))
        hb, nm = _tc_upd(np_pad, aggp, invd, hb,
                         W_upd[i], b_upd[i].reshape(1, _DH),
                         ln_g[i].reshape(1, _DH), ln_b[i].reshape(1, _DH),
                         W_msg[(i + 1) % l_layers])
        if i % 2 == 1:
            outs.append(hb)

    out = _tc_final(np_pad, outs[0], outs[1], outs[2], outs[3],
                    W_out1, b_out1.reshape(1, _DH), W_out2,
                    b_out2.reshape(1, _DH))
    return out[:n]
